# baseline (speedup 1.0000x reference)
import functools

import jax
import jax.numpy as jnp
from jax import lax
from jax.experimental import pallas as pl
from jax.experimental.pallas import tpu as pltpu
from jax.experimental.pallas import tpu_sc as plsc

N = 10000
E = 320000
D = 128
G = 64
C = 10

NC = 2
NS = 16

CH = 80
NCHUNK = 256
EPT = NCHUNK * CH
EBUF = EPT + 2 * CH
EPAD = NS * EPT
OWN = 5120
ACC_ROWS = 5128
WROWS = OWN // NS

_sc_mesh = plsc.VectorSubcoreMesh(
    core_axis_name="c", subcore_axis_name="s", num_cores=NC, num_subcores=NS)


@functools.partial(
    pl.kernel,
    out_type=jax.ShapeDtypeStruct((NC, OWN, D), jnp.float32),
    mesh=_sc_mesh,
    compiler_params=pltpu.CompilerParams(needs_layout_passes=False),
    scratch_types=[
        pltpu.VMEM((EPT // 4,), jnp.int32),
        pltpu.VMEM((EPT // 4,), jnp.int32),
        pltpu.VMEM((EBUF,), jnp.int32),
        pltpu.VMEM((NCHUNK + 2, CH), jnp.int32),
        pltpu.VMEM((2, CH, D), jnp.float32),
        pltpu.VMEM_SHARED((ACC_ROWS, D), jnp.float32),
        [pltpu.SemaphoreType.DMA] * 2,
    ],
)
def _sc_edge_agg(x_hbm, src_hbm, dst_hbm, zeros_hbm, out_hbm,
                 src_v, dst_v, srcl, dstl, rows, acc, sems):
    c = lax.axis_index("c")
    s = lax.axis_index("s")

    pltpu.sync_copy(zeros_hbm, acc.at[pl.ds(s * WROWS, WROWS)])

    @pl.when(s == 0)
    def _():
        pltpu.sync_copy(zeros_hbm.at[pl.ds(0, 8)], acc.at[pl.ds(OWN, 8)])

    base = c * OWN
    q = EPT // 4

    m = jnp.int32(0)
    for p in range(4):
        pltpu.sync_copy(src_hbm.at[s, pl.ds(p * q, q)], src_v)
        pltpu.sync_copy(dst_hbm.at[s, pl.ds(p * q, q)], dst_v)

        def compact(i, off):
            vs = src_v[pl.ds(16 * i, 16)]
            t = dst_v[pl.ds(16 * i, 16)] - base
            ok = (t >= 0) & (t < OWN)
            oki = ok.astype(jnp.int32)
            pos = off + plsc.cumsum(oki) - oki
            plsc.store_scatter(srcl, [pos], vs, mask=ok)
            plsc.store_scatter(dstl, [pos // CH, pos % CH], t, mask=ok)
            return off + jnp.sum(oki)

        m = lax.fori_loop(0, q // 16, compact, m)

    lanes = lax.iota(jnp.int32, 16)
    full = jnp.full((16,), True)
    for k in range(2 * CH // 16):
        pos = m + 16 * k + lanes
        plsc.store_scatter(srcl, [pos], jnp.zeros((16,), jnp.int32), mask=full)
        plsc.store_scatter(dstl, [pos // CH, pos % CH],
                           jnp.full((16,), OWN, jnp.int32), mask=full)

    plsc.subcore_barrier()

    npairs = jnp.maximum((m + 2 * CH - 1) // (2 * CH), 1)

    for b in range(2):
        pltpu.async_copy(x_hbm.at[srcl.at[pl.ds(b * CH, CH)]], rows.at[b], sems[b])

    def body(g, _):
        for b in range(2):
            j = 2 * g + b
            pltpu.make_async_copy(
                x_hbm.at[srcl.at[pl.ds(j * CH, CH)]], rows.at[b], sems[b]).wait()
            pltpu.sync_copy(rows.at[b], acc.at[dstl.at[j]], add=True)

            @pl.when(g < npairs - 1)
            def _():
                pltpu.async_copy(
                    x_hbm.at[srcl.at[pl.ds((j + 2) * CH, CH)]], rows.at[b], sems[b])
        return _

    lax.fori_loop(0, npairs, body, None)

    plsc.subcore_barrier()

    pltpu.sync_copy(acc.at[pl.ds(s * WROWS, WROWS)],
                    out_hbm.at[c].at[pl.ds(s * WROWS, WROWS)])


BLK = 2000
GRID = N // BLK


def _mlp1_body(x_ref, a_ref, wa_ref, ba_ref, wb_ref, bb_ref, o_ref):
    h = x_ref[...] + a_ref[...]
    h = jnp.maximum(
        jnp.dot(h, wa_ref[...], preferred_element_type=jnp.float32) + ba_ref[...], 0.0)
    o_ref[...] = jnp.maximum(
        jnp.dot(h, wb_ref[...], preferred_element_type=jnp.float32) + bb_ref[...], 0.0)


def _mlp2_pool_body(h_ref, a_ref, wa_ref, ba_ref, wb_ref, bb_ref,
                    batch_ref, wfc_ref, bfc_ref, o_ref, sums_ref, cnts_ref):
    i = pl.program_id(0)

    @pl.when(i == 0)
    def _():
        sums_ref[...] = jnp.zeros_like(sums_ref)
        cnts_ref[...] = jnp.zeros_like(cnts_ref)

    h = h_ref[...] + a_ref[...]
    h = jnp.maximum(
        jnp.dot(h, wa_ref[...], preferred_element_type=jnp.float32) + ba_ref[...], 0.0)
    h = jnp.maximum(
        jnp.dot(h, wb_ref[...], preferred_element_type=jnp.float32) + bb_ref[...], 0.0)

    b = batch_ref[0, 0, :]
    oh = (lax.broadcasted_iota(jnp.int32, (G, BLK), 0) == b[None, :]).astype(jnp.float32)
    sums_ref[...] += jnp.dot(oh, h, preferred_element_type=jnp.float32)
    cnts_ref[...] += jnp.broadcast_to(jnp.sum(oh, axis=1, keepdims=True), (G, D))

    @pl.when(i == GRID - 1)
    def _():
        pooled = sums_ref[...] / jnp.maximum(cnts_ref[...], 1.0)
        logits = jnp.dot(pooled, wfc_ref[...], preferred_element_type=jnp.float32)
        o_ref[...] = jax.nn.sigmoid(logits + bfc_ref[...])


def _row_spec():
    return pl.BlockSpec((BLK, D), lambda i: (i, 0))


def _w_spec():
    return pl.BlockSpec((D, D), lambda i: (0, 0))


def _b_spec():
    return pl.BlockSpec((1, D), lambda i: (0, 0))


def kernel(x, edge_index, batch, W1a, b1a, W1b, b1b, W2a, b2a, W2b, b2b, Wfc, bfc):
    src = edge_index[0]
    dst = edge_index[1]
    pad = EPAD - E
    src_p = jnp.concatenate([src, jnp.zeros((pad,), jnp.int32)]).reshape(NS, EPT)
    dst_p = jnp.concatenate([dst, jnp.full((pad,), NC * OWN, jnp.int32)]).reshape(NS, EPT)
    zeros_tile = jnp.zeros((WROWS, D), jnp.float32)

    agg1 = _sc_edge_agg(x, src_p, dst_p, zeros_tile).reshape(NC * OWN, D)

    mlp1 = pl.pallas_call(
        _mlp1_body,
        grid=(GRID,),
        in_specs=[_row_spec(), _row_spec(), _w_spec(), _b_spec(), _w_spec(), _b_spec()],
        out_specs=_row_spec(),
        out_shape=jax.ShapeDtypeStruct((N, D), jnp.float32),
    )
    h1 = mlp1(x, agg1, W1a, b1a.reshape(1, D), W1b, b1b.reshape(1, D))

    agg2 = _sc_edge_agg(h1, src_p, dst_p, zeros_tile).reshape(NC * OWN, D)

    batch3 = batch.reshape(GRID, 1, BLK)
    wfc_pad = jnp.pad(Wfc, ((0, 0), (0, D - C)))
    bfc_pad = jnp.pad(bfc, (0, D - C)).reshape(1, D)

    mlp2 = pl.pallas_call(
        _mlp2_pool_body,
        grid=(GRID,),
        in_specs=[
            _row_spec(), _row_spec(), _w_spec(), _b_spec(), _w_spec(), _b_spec(),
            pl.BlockSpec((1, 1, BLK), lambda i: (i, 0, 0)),
            _w_spec(), _b_spec(),
        ],
        out_specs=pl.BlockSpec((G, D), lambda i: (0, 0)),
        out_shape=jax.ShapeDtypeStruct((G, D), jnp.float32),
        scratch_shapes=[
            pltpu.VMEM((G, D), jnp.float32),
            pltpu.VMEM((G, D), jnp.float32),
        ],
    )
    out = mlp2(h1, agg2, W2a, b2a.reshape(1, D), W2b, b2b.reshape(1, D),
               batch3, wfc_pad, bfc_pad)
    return out[:, :C]


# speedup vs baseline: 1.2318x; 1.2318x over previous
import functools

import jax
import jax.numpy as jnp
from jax import lax
from jax.experimental import pallas as pl
from jax.experimental.pallas import tpu as pltpu
from jax.experimental.pallas import tpu_sc as plsc

N = 10000
E = 320000
D = 128
G = 64
C = 10

NC = 2
NS = 16

CH = 64
NCHUNK = 320
EPT = NCHUNK * CH
EBUF = EPT + 2 * CH
EPAD = NS * EPT
OWN = 5120
ACC_ROWS = 5128
WROWS = OWN // NS

_sc_mesh = plsc.VectorSubcoreMesh(
    core_axis_name="c", subcore_axis_name="s", num_cores=NC, num_subcores=NS)


@functools.partial(
    pl.kernel,
    out_type=jax.ShapeDtypeStruct((NC, OWN, D), jnp.float32),
    mesh=_sc_mesh,
    compiler_params=pltpu.CompilerParams(needs_layout_passes=False),
    scratch_types=[
        pltpu.VMEM((EPT // 4,), jnp.int32),
        pltpu.VMEM((EPT // 4,), jnp.int32),
        pltpu.VMEM((EBUF,), jnp.int32),
        pltpu.VMEM((NCHUNK + 2, CH), jnp.int32),
        pltpu.VMEM((2, CH, D), jnp.float32),
        pltpu.VMEM_SHARED((ACC_ROWS, D), jnp.float32),
        [pltpu.SemaphoreType.DMA] * 2,
    ],
)
def _sc_edge_agg(x_hbm, src_hbm, dst_hbm, zeros_hbm, out_hbm,
                 src_v, dst_v, srcl, dstl, rows, acc, sems):
    c = lax.axis_index("c")
    s = lax.axis_index("s")

    pltpu.sync_copy(zeros_hbm, acc.at[pl.ds(s * WROWS, WROWS)])

    @pl.when(s == 0)
    def _():
        pltpu.sync_copy(zeros_hbm.at[pl.ds(0, 8)], acc.at[pl.ds(OWN, 8)])

    base = c * OWN
    q = EPT // 4

    m = jnp.int32(0)
    for p in range(4):
        pltpu.sync_copy(src_hbm.at[s, pl.ds(p * q, q)], src_v)
        pltpu.sync_copy(dst_hbm.at[s, pl.ds(p * q, q)], dst_v)

        def compact(i, off):
            vs = src_v[pl.ds(16 * i, 16)]
            t = dst_v[pl.ds(16 * i, 16)] - base
            ok = (t >= 0) & (t < OWN)
            oki = ok.astype(jnp.int32)
            pos = off + plsc.cumsum(oki) - oki
            plsc.store_scatter(srcl, [pos], vs, mask=ok)
            plsc.store_scatter(dstl, [pos // CH, pos % CH], t, mask=ok)
            return off + jnp.sum(oki)

        m = lax.fori_loop(0, q // 16, compact, m)

    lanes = lax.iota(jnp.int32, 16)
    full = jnp.full((16,), True)
    for k in range(2 * CH // 16):
        pos = m + 16 * k + lanes
        plsc.store_scatter(srcl, [pos], jnp.zeros((16,), jnp.int32), mask=full)
        plsc.store_scatter(dstl, [pos // CH, pos % CH],
                           jnp.full((16,), OWN, jnp.int32), mask=full)

    plsc.subcore_barrier()

    npairs = jnp.maximum((m + 2 * CH - 1) // (2 * CH), 1)

    for b in range(2):
        pltpu.async_copy(x_hbm.at[srcl.at[pl.ds(b * CH, CH)]], rows.at[b], sems[b])

    def body(g, _):
        for b in range(2):
            j = 2 * g + b
            pltpu.make_async_copy(
                x_hbm.at[srcl.at[pl.ds(j * CH, CH)]], rows.at[b], sems[b]).wait()
            pltpu.sync_copy(rows.at[b], acc.at[dstl.at[j]], add=True)

            @pl.when(g < npairs - 1)
            def _():
                pltpu.async_copy(
                    x_hbm.at[srcl.at[pl.ds((j + 2) * CH, CH)]], rows.at[b], sems[b])
        return _

    lax.fori_loop(0, npairs, body, None)

    plsc.subcore_barrier()

    pltpu.sync_copy(acc.at[pl.ds(s * WROWS, WROWS)],
                    out_hbm.at[c].at[pl.ds(s * WROWS, WROWS)])


BLK = 2000
GRID = N // BLK


def _mlp1_body(x_ref, a_ref, wa_ref, ba_ref, wb_ref, bb_ref, o_ref):
    h = x_ref[...] + a_ref[...]
    h = jnp.maximum(
        jnp.dot(h, wa_ref[...], preferred_element_type=jnp.float32) + ba_ref[...], 0.0)
    o_ref[...] = jnp.maximum(
        jnp.dot(h, wb_ref[...], preferred_element_type=jnp.float32) + bb_ref[...], 0.0)


def _mlp2_pool_body(h_ref, a_ref, wa_ref, ba_ref, wb_ref, bb_ref,
                    batch_ref, wfc_ref, bfc_ref, o_ref, sums_ref, cnts_ref):
    i = pl.program_id(0)

    @pl.when(i == 0)
    def _():
        sums_ref[...] = jnp.zeros_like(sums_ref)
        cnts_ref[...] = jnp.zeros_like(cnts_ref)

    h = h_ref[...] + a_ref[...]
    h = jnp.maximum(
        jnp.dot(h, wa_ref[...], preferred_element_type=jnp.float32) + ba_ref[...], 0.0)
    h = jnp.maximum(
        jnp.dot(h, wb_ref[...], preferred_element_type=jnp.float32) + bb_ref[...], 0.0)

    b = batch_ref[0, 0, :]
    oh = (lax.broadcasted_iota(jnp.int32, (G, BLK), 0) == b[None, :]).astype(jnp.float32)
    sums_ref[...] += jnp.dot(oh, h, preferred_element_type=jnp.float32)
    cnts_ref[...] += jnp.broadcast_to(jnp.sum(oh, axis=1, keepdims=True), (G, D))

    @pl.when(i == GRID - 1)
    def _():
        pooled = sums_ref[...] / jnp.maximum(cnts_ref[...], 1.0)
        logits = jnp.dot(pooled, wfc_ref[...], preferred_element_type=jnp.float32)
        o_ref[...] = jax.nn.sigmoid(logits + bfc_ref[...])


def _row_spec():
    return pl.BlockSpec((BLK, D), lambda i: (i, 0))


def _w_spec():
    return pl.BlockSpec((D, D), lambda i: (0, 0))


def _b_spec():
    return pl.BlockSpec((1, D), lambda i: (0, 0))


def kernel(x, edge_index, batch, W1a, b1a, W1b, b1b, W2a, b2a, W2b, b2b, Wfc, bfc):
    src = edge_index[0]
    dst = edge_index[1]
    pad = EPAD - E
    src_p = jnp.concatenate([src, jnp.zeros((pad,), jnp.int32)]).reshape(NS, EPT)
    dst_p = jnp.concatenate([dst, jnp.full((pad,), NC * OWN, jnp.int32)]).reshape(NS, EPT)
    zeros_tile = jnp.zeros((WROWS, D), jnp.float32)

    agg1 = _sc_edge_agg(x, src_p, dst_p, zeros_tile).reshape(NC * OWN, D)

    mlp1 = pl.pallas_call(
        _mlp1_body,
        grid=(GRID,),
        in_specs=[_row_spec(), _row_spec(), _w_spec(), _b_spec(), _w_spec(), _b_spec()],
        out_specs=_row_spec(),
        out_shape=jax.ShapeDtypeStruct((N, D), jnp.float32),
    )
    h1 = mlp1(x, agg1, W1a, b1a.reshape(1, D), W1b, b1b.reshape(1, D))

    agg2 = _sc_edge_agg(h1, src_p, dst_p, zeros_tile).reshape(NC * OWN, D)

    batch3 = batch.reshape(GRID, 1, BLK)
    wfc_pad = jnp.pad(Wfc, ((0, 0), (0, D - C)))
    bfc_pad = jnp.pad(bfc, (0, D - C)).reshape(1, D)

    mlp2 = pl.pallas_call(
        _mlp2_pool_body,
        grid=(GRID,),
        in_specs=[
            _row_spec(), _row_spec(), _w_spec(), _b_spec(), _w_spec(), _b_spec(),
            pl.BlockSpec((1, 1, BLK), lambda i: (i, 0, 0)),
            _w_spec(), _b_spec(),
        ],
        out_specs=pl.BlockSpec((G, D), lambda i: (0, 0)),
        out_shape=jax.ShapeDtypeStruct((G, D), jnp.float32),
        scratch_shapes=[
            pltpu.VMEM((G, D), jnp.float32),
            pltpu.VMEM((G, D), jnp.float32),
        ],
    )
    out = mlp2(h1, agg2, W2a, b2a.reshape(1, D), W2b, b2b.reshape(1, D),
               batch3, wfc_pad, bfc_pad)
    return out[:, :C]
